# baseline (speedup 1.0000x reference)
import functools

import jax
import jax.numpy as jnp
from jax import lax
from jax.experimental import pallas as pl
from jax.experimental.pallas import tpu as pltpu
from jax.experimental.pallas import tpu_sc as plsc

N_NODES = 10000
D_FEAT = 128
N_EDGES = 320000
DEPTH = 3

NC = 2
NS = 16
NW = NC * NS

EDGES_PER_TILE = N_EDGES // NW
CHUNK = 80
CHUNKS = EDGES_PER_TILE // CHUNK
N_PAD = 10112
ROWS_PER_TILE = N_PAD // NS

_mesh = plsc.VectorSubcoreMesh(core_axis_name="c", subcore_axis_name="s")


def _sc_body(x_hbm, src_hbm, dst_hbm, zeros_hbm, out_hbm,
             act_sh, src_v, dst_v, rows0, rows1, gsem0, gsem1):
    c = lax.axis_index("c")
    s = lax.axis_index("s")
    wid = c * NS + s

    pltpu.sync_copy(zeros_hbm, act_sh.at[pl.ds(s * ROWS_PER_TILE, ROWS_PER_TILE)])
    pltpu.sync_copy(src_hbm.at[wid], src_v)
    pltpu.sync_copy(dst_hbm.at[wid], dst_v)
    plsc.subcore_barrier()

    def body(j, carry):
        pltpu.async_copy(x_hbm.at[src_v.at[j]], rows0, gsem0).wait()
        pltpu.sync_copy(rows0, act_sh.at[dst_v.at[j]], add=True)
        return carry

    lax.fori_loop(0, CHUNKS, body, 0)
    plsc.subcore_barrier()
    sl = pl.ds(s * ROWS_PER_TILE, ROWS_PER_TILE)
    pltpu.sync_copy(act_sh.at[sl], out_hbm.at[c, sl])


_sc_scatter = pl.kernel(
    _sc_body,
    out_type=jax.ShapeDtypeStruct((NC, N_PAD, D_FEAT), jnp.float32),
    mesh=_mesh,
    compiler_params=pltpu.CompilerParams(use_tc_tiling_on_sc=False),
    scratch_types=[
        pltpu.VMEM_SHARED((N_PAD, D_FEAT), jnp.float32),
        pltpu.VMEM((CHUNKS, CHUNK), jnp.int32),
        pltpu.VMEM((CHUNKS, CHUNK), jnp.int32),
        pltpu.VMEM((CHUNK, D_FEAT), jnp.float32),
        pltpu.VMEM((CHUNK, D_FEAT), jnp.float32),
        pltpu.SemaphoreType.DMA,
        pltpu.SemaphoreType.DMA,
    ],
)


def _tc_body(p_ref, x_ref, w1_ref, w2_ref, b_ref, o_ref):
    act = p_ref[0] + p_ref[1]
    o_ref[...] = jnp.maximum(
        jnp.dot(act, w1_ref[...], preferred_element_type=jnp.float32)
        + jnp.dot(x_ref[...], w2_ref[...], preferred_element_type=jnp.float32)
        + b_ref[...],
        0.0,
    )


_BR = 2000


@jax.jit
def _tc_update(parts, x, w1, w2, b2):
    return pl.pallas_call(
        _tc_body,
        grid=(N_NODES // _BR,),
        in_specs=[
            pl.BlockSpec((NC, _BR, D_FEAT), lambda i: (0, i, 0)),
            pl.BlockSpec((_BR, D_FEAT), lambda i: (i, 0)),
            pl.BlockSpec((D_FEAT, D_FEAT), lambda i: (0, 0)),
            pl.BlockSpec((D_FEAT, D_FEAT), lambda i: (0, 0)),
            pl.BlockSpec((1, D_FEAT), lambda i: (0, 0)),
        ],
        out_specs=pl.BlockSpec((_BR, D_FEAT), lambda i: (i, 0)),
        out_shape=jax.ShapeDtypeStruct((N_NODES, D_FEAT), jnp.float32),
    )(parts, x, w1, w2, b2)


def kernel(node_embedding, incidence, W, b):
    src = incidence[1].reshape(NW, CHUNKS, CHUNK)
    dst = incidence[0].reshape(NW, CHUNKS, CHUNK)
    w1 = W[:D_FEAT]
    w2 = W[D_FEAT:]
    b2 = b.reshape(1, D_FEAT)
    zeros = jnp.zeros((ROWS_PER_TILE, D_FEAT), jnp.float32)
    x = node_embedding
    for _ in range(DEPTH):
        parts = _sc_scatter(x, src, dst, zeros)
        x = _tc_update(parts, x, w1, w2, b2)
    return x


# speedup vs baseline: 1.7022x; 1.7022x over previous
import functools

import jax
import jax.numpy as jnp
from jax import lax
from jax.experimental import pallas as pl
from jax.experimental.pallas import tpu as pltpu
from jax.experimental.pallas import tpu_sc as plsc

N_NODES = 10000
D_FEAT = 128
N_EDGES = 320000
DEPTH = 3

NC = 2
NS = 16
NW = NC * NS

EDGES_PER_TILE = N_EDGES // NW
CHUNK = 100
CHUNKS = EDGES_PER_TILE // CHUNK
N_PAD = 10112
ROWS_PER_TILE = N_PAD // NS

_mesh = plsc.VectorSubcoreMesh(core_axis_name="c", subcore_axis_name="s")


def _sc_body(x_hbm, src_hbm, dst_hbm, zeros_hbm, out_hbm,
             act_sh, src_v, dst_v, rows0, rows1, gsem0, gsem1):
    c = lax.axis_index("c")
    s = lax.axis_index("s")
    wid = c * NS + s

    pltpu.sync_copy(zeros_hbm, act_sh.at[pl.ds(s * ROWS_PER_TILE, ROWS_PER_TILE)])
    pltpu.sync_copy(src_hbm.at[wid], src_v)
    pltpu.sync_copy(dst_hbm.at[wid], dst_v)
    plsc.subcore_barrier()

    nh = CHUNKS // 2
    pltpu.async_copy(x_hbm.at[src_v.at[0]], rows0, gsem0)
    pltpu.async_copy(x_hbm.at[src_v.at[1]], rows1, gsem1)

    def body(i, carry):
        j0 = 2 * i
        j1 = j0 + 1
        pltpu.make_async_copy(x_hbm.at[src_v.at[j0]], rows0, gsem0).wait()
        pltpu.sync_copy(rows0, act_sh.at[dst_v.at[j0]], add=True)
        pltpu.async_copy(x_hbm.at[src_v.at[j0 + 2]], rows0, gsem0)
        pltpu.make_async_copy(x_hbm.at[src_v.at[j1]], rows1, gsem1).wait()
        pltpu.sync_copy(rows1, act_sh.at[dst_v.at[j1]], add=True)
        pltpu.async_copy(x_hbm.at[src_v.at[j1 + 2]], rows1, gsem1)
        return carry

    lax.fori_loop(0, nh - 1, body, 0)
    pltpu.make_async_copy(x_hbm.at[src_v.at[CHUNKS - 2]], rows0, gsem0).wait()
    pltpu.sync_copy(rows0, act_sh.at[dst_v.at[CHUNKS - 2]], add=True)
    pltpu.make_async_copy(x_hbm.at[src_v.at[CHUNKS - 1]], rows1, gsem1).wait()
    pltpu.sync_copy(rows1, act_sh.at[dst_v.at[CHUNKS - 1]], add=True)
    plsc.subcore_barrier()
    sl = pl.ds(s * ROWS_PER_TILE, ROWS_PER_TILE)
    pltpu.sync_copy(act_sh.at[sl], out_hbm.at[c, sl])


_sc_scatter = pl.kernel(
    _sc_body,
    out_type=jax.ShapeDtypeStruct((NC, N_PAD, D_FEAT), jnp.float32),
    mesh=_mesh,
    compiler_params=pltpu.CompilerParams(use_tc_tiling_on_sc=False),
    scratch_types=[
        pltpu.VMEM_SHARED((N_PAD, D_FEAT), jnp.float32),
        pltpu.VMEM((CHUNKS, CHUNK), jnp.int32),
        pltpu.VMEM((CHUNKS, CHUNK), jnp.int32),
        pltpu.VMEM((CHUNK, D_FEAT), jnp.float32),
        pltpu.VMEM((CHUNK, D_FEAT), jnp.float32),
        pltpu.SemaphoreType.DMA,
        pltpu.SemaphoreType.DMA,
    ],
)


def _tc_body(p_ref, x_ref, w1_ref, w2_ref, b_ref, o_ref):
    act = p_ref[0] + p_ref[1]
    o_ref[...] = jnp.maximum(
        jnp.dot(act, w1_ref[...], preferred_element_type=jnp.float32)
        + jnp.dot(x_ref[...], w2_ref[...], preferred_element_type=jnp.float32)
        + b_ref[...],
        0.0,
    )


_BR = 2000


@jax.jit
def _tc_update(parts, x, w1, w2, b2):
    return pl.pallas_call(
        _tc_body,
        grid=(N_NODES // _BR,),
        in_specs=[
            pl.BlockSpec((NC, _BR, D_FEAT), lambda i: (0, i, 0)),
            pl.BlockSpec((_BR, D_FEAT), lambda i: (i, 0)),
            pl.BlockSpec((D_FEAT, D_FEAT), lambda i: (0, 0)),
            pl.BlockSpec((D_FEAT, D_FEAT), lambda i: (0, 0)),
            pl.BlockSpec((1, D_FEAT), lambda i: (0, 0)),
        ],
        out_specs=pl.BlockSpec((_BR, D_FEAT), lambda i: (i, 0)),
        out_shape=jax.ShapeDtypeStruct((N_NODES, D_FEAT), jnp.float32),
    )(parts, x, w1, w2, b2)


def kernel(node_embedding, incidence, W, b):
    src = incidence[1].reshape(NW, CHUNKS, CHUNK)
    dst = incidence[0].reshape(NW, CHUNKS, CHUNK)
    w1 = W[:D_FEAT]
    w2 = W[D_FEAT:]
    b2 = b.reshape(1, D_FEAT)
    zeros = jnp.zeros((ROWS_PER_TILE, D_FEAT), jnp.float32)
    x = node_embedding
    for _ in range(DEPTH):
        parts = _sc_scatter(x, src, dst, zeros)
        x = _tc_update(parts, x, w1, w2, b2)
    return x


# speedup vs baseline: 1.8306x; 1.0754x over previous
import functools

import jax
import jax.numpy as jnp
from jax import lax
from jax.experimental import pallas as pl
from jax.experimental.pallas import tpu as pltpu
from jax.experimental.pallas import tpu_sc as plsc

N_NODES = 10000
D_FEAT = 128
N_EDGES = 320000
DEPTH = 3

NC = 2
NS = 16
NW = NC * NS

EDGES_PER_TILE = N_EDGES // NW
CHUNK = 50
CHUNKS = EDGES_PER_TILE // CHUNK
NBUF = 4
N_PAD = 10112
ROWS_PER_TILE = N_PAD // NS

_mesh = plsc.VectorSubcoreMesh(core_axis_name="c", subcore_axis_name="s")


def _sc_body(x_hbm, src_hbm, dst_hbm, zeros_hbm, out_hbm,
             act_sh, src_v, dst_v, rows0, rows1, rows2, rows3,
             gsem0, gsem1, gsem2, gsem3):
    c = lax.axis_index("c")
    s = lax.axis_index("s")
    wid = c * NS + s

    pltpu.sync_copy(zeros_hbm, act_sh.at[pl.ds(s * ROWS_PER_TILE, ROWS_PER_TILE)])
    pltpu.sync_copy(src_hbm.at[wid], src_v)
    pltpu.sync_copy(dst_hbm.at[wid], dst_v)
    plsc.subcore_barrier()

    rows = [rows0, rows1, rows2, rows3]
    gsems = [gsem0, gsem1, gsem2, gsem3]
    for k in range(NBUF):
        pltpu.async_copy(x_hbm.at[src_v.at[k]], rows[k], gsems[k])

    def body(i, carry):
        j = NBUF * i
        for k in range(NBUF):
            pltpu.make_async_copy(x_hbm.at[src_v.at[j + k]], rows[k], gsems[k]).wait()
            pltpu.sync_copy(rows[k], act_sh.at[dst_v.at[j + k]], add=True)
            pltpu.async_copy(x_hbm.at[src_v.at[j + k + NBUF]], rows[k], gsems[k])
        return carry

    lax.fori_loop(0, CHUNKS // NBUF - 1, body, 0)
    for k in range(NBUF):
        j = CHUNKS - NBUF + k
        pltpu.make_async_copy(x_hbm.at[src_v.at[j]], rows[k], gsems[k]).wait()
        pltpu.sync_copy(rows[k], act_sh.at[dst_v.at[j]], add=True)
    plsc.subcore_barrier()
    sl = pl.ds(s * ROWS_PER_TILE, ROWS_PER_TILE)
    pltpu.sync_copy(act_sh.at[sl], out_hbm.at[c, sl])


_sc_scatter = pl.kernel(
    _sc_body,
    out_type=jax.ShapeDtypeStruct((NC, N_PAD, D_FEAT), jnp.float32),
    mesh=_mesh,
    compiler_params=pltpu.CompilerParams(use_tc_tiling_on_sc=False),
    scratch_types=[
        pltpu.VMEM_SHARED((N_PAD, D_FEAT), jnp.float32),
        pltpu.VMEM((CHUNKS, CHUNK), jnp.int32),
        pltpu.VMEM((CHUNKS, CHUNK), jnp.int32),
        pltpu.VMEM((CHUNK, D_FEAT), jnp.float32),
        pltpu.VMEM((CHUNK, D_FEAT), jnp.float32),
        pltpu.VMEM((CHUNK, D_FEAT), jnp.float32),
        pltpu.VMEM((CHUNK, D_FEAT), jnp.float32),
        pltpu.SemaphoreType.DMA,
        pltpu.SemaphoreType.DMA,
        pltpu.SemaphoreType.DMA,
        pltpu.SemaphoreType.DMA,
    ],
)


def _tc_body(p_ref, x_ref, w1_ref, w2_ref, b_ref, o_ref):
    act = p_ref[0] + p_ref[1]
    o_ref[...] = jnp.maximum(
        jnp.dot(act, w1_ref[...], preferred_element_type=jnp.float32)
        + jnp.dot(x_ref[...], w2_ref[...], preferred_element_type=jnp.float32)
        + b_ref[...],
        0.0,
    )


_BR = 2000


@jax.jit
def _tc_update(parts, x, w1, w2, b2):
    return pl.pallas_call(
        _tc_body,
        grid=(N_NODES // _BR,),
        in_specs=[
            pl.BlockSpec((NC, _BR, D_FEAT), lambda i: (0, i, 0)),
            pl.BlockSpec((_BR, D_FEAT), lambda i: (i, 0)),
            pl.BlockSpec((D_FEAT, D_FEAT), lambda i: (0, 0)),
            pl.BlockSpec((D_FEAT, D_FEAT), lambda i: (0, 0)),
            pl.BlockSpec((1, D_FEAT), lambda i: (0, 0)),
        ],
        out_specs=pl.BlockSpec((_BR, D_FEAT), lambda i: (i, 0)),
        out_shape=jax.ShapeDtypeStruct((N_NODES, D_FEAT), jnp.float32),
    )(parts, x, w1, w2, b2)


def kernel(node_embedding, incidence, W, b):
    src = incidence[1].reshape(NW, CHUNKS, CHUNK)
    dst = incidence[0].reshape(NW, CHUNKS, CHUNK)
    w1 = W[:D_FEAT]
    w2 = W[D_FEAT:]
    b2 = b.reshape(1, D_FEAT)
    zeros = jnp.zeros((ROWS_PER_TILE, D_FEAT), jnp.float32)
    x = node_embedding
    for _ in range(DEPTH):
        parts = _sc_scatter(x, src, dst, zeros)
        x = _tc_update(parts, x, w1, w2, b2)
    return x


# speedup vs baseline: 1.8335x; 1.0016x over previous
import functools

import jax
import jax.numpy as jnp
from jax import lax
from jax.experimental import pallas as pl
from jax.experimental.pallas import tpu as pltpu
from jax.experimental.pallas import tpu_sc as plsc

N_NODES = 10000
D_FEAT = 128
N_EDGES = 320000
DEPTH = 3

NC = 2
NS = 16
NW = NC * NS

EDGES_PER_TILE = N_EDGES // NW
CHUNK = 50
CHUNKS = EDGES_PER_TILE // CHUNK
NBUF = 4
N_PAD = 10000
ROWS_PER_TILE = N_PAD // NS
ZROWS = 125

_mesh = plsc.VectorSubcoreMesh(core_axis_name="c", subcore_axis_name="s")


def _sc_body(x_hbm, src_hbm, dst_hbm, zeros_hbm, out_hbm,
             act_sh, src_v, dst_v, rows0, rows1, rows2, rows3,
             gsem0, gsem1, gsem2, gsem3):
    c = lax.axis_index("c")
    s = lax.axis_index("s")
    wid = c * NS + s

    pltpu.sync_copy(zeros_hbm, act_sh.at[pl.ds(s * ROWS_PER_TILE, ROWS_PER_TILE)])
    pltpu.sync_copy(src_hbm.at[wid], src_v)
    pltpu.sync_copy(dst_hbm.at[wid], dst_v)
    plsc.subcore_barrier()

    rows = [rows0, rows1, rows2, rows3]
    gsems = [gsem0, gsem1, gsem2, gsem3]
    for k in range(NBUF):
        pltpu.async_copy(x_hbm.at[src_v.at[k]], rows[k], gsems[k])

    def body(i, carry):
        j = NBUF * i
        for k in range(NBUF):
            pltpu.make_async_copy(x_hbm.at[src_v.at[j + k]], rows[k], gsems[k]).wait()
            pltpu.sync_copy(rows[k], act_sh.at[dst_v.at[j + k]], add=True)
            pltpu.async_copy(x_hbm.at[src_v.at[j + k + NBUF]], rows[k], gsems[k])
        return carry

    lax.fori_loop(0, CHUNKS // NBUF - 1, body, 0)
    for k in range(NBUF):
        j = CHUNKS - NBUF + k
        pltpu.make_async_copy(x_hbm.at[src_v.at[j]], rows[k], gsems[k]).wait()
        pltpu.sync_copy(rows[k], act_sh.at[dst_v.at[j]], add=True)
    plsc.subcore_barrier()
    sl = pl.ds(s * ROWS_PER_TILE, ROWS_PER_TILE)
    pltpu.sync_copy(act_sh.at[sl], out_hbm.at[c, sl])


_sc_scatter = pl.kernel(
    _sc_body,
    out_type=jax.ShapeDtypeStruct((NC, N_PAD, D_FEAT), jnp.float32),
    mesh=_mesh,
    compiler_params=pltpu.CompilerParams(use_tc_tiling_on_sc=False),
    scratch_types=[
        pltpu.VMEM_SHARED((N_PAD, D_FEAT), jnp.float32),
        pltpu.VMEM((CHUNKS, CHUNK), jnp.int32),
        pltpu.VMEM((CHUNKS, CHUNK), jnp.int32),
        pltpu.VMEM((CHUNK, D_FEAT), jnp.float32),
        pltpu.VMEM((CHUNK, D_FEAT), jnp.float32),
        pltpu.VMEM((CHUNK, D_FEAT), jnp.float32),
        pltpu.VMEM((CHUNK, D_FEAT), jnp.float32),
        pltpu.SemaphoreType.DMA,
        pltpu.SemaphoreType.DMA,
        pltpu.SemaphoreType.DMA,
        pltpu.SemaphoreType.DMA,
    ],
)


def _tc_body(p_ref, x_ref, w1_ref, w2_ref, b_ref, o_ref):
    act = p_ref[0] + p_ref[1]
    o_ref[...] = jnp.maximum(
        jnp.dot(act, w1_ref[...], preferred_element_type=jnp.float32)
        + jnp.dot(x_ref[...], w2_ref[...], preferred_element_type=jnp.float32)
        + b_ref[...],
        0.0,
    )


_BR = 2000


@jax.jit
def _tc_update(parts, x, w1, w2, b2):
    return pl.pallas_call(
        _tc_body,
        grid=(N_NODES // _BR,),
        in_specs=[
            pl.BlockSpec((NC, _BR, D_FEAT), lambda i: (0, i, 0)),
            pl.BlockSpec((_BR, D_FEAT), lambda i: (i, 0)),
            pl.BlockSpec((D_FEAT, D_FEAT), lambda i: (0, 0)),
            pl.BlockSpec((D_FEAT, D_FEAT), lambda i: (0, 0)),
            pl.BlockSpec((1, D_FEAT), lambda i: (0, 0)),
        ],
        out_specs=pl.BlockSpec((_BR, D_FEAT), lambda i: (i, 0)),
        out_shape=jax.ShapeDtypeStruct((N_NODES, D_FEAT), jnp.float32),
    )(parts, x, w1, w2, b2)


def kernel(node_embedding, incidence, W, b):
    src = incidence[1].reshape(NW, CHUNKS, CHUNK)
    dst = incidence[0].reshape(NW, CHUNKS, CHUNK)
    w1 = W[:D_FEAT]
    w2 = W[D_FEAT:]
    b2 = b.reshape(1, D_FEAT)
    zeros = jnp.zeros((ROWS_PER_TILE, D_FEAT), jnp.float32)
    x = node_embedding
    for _ in range(DEPTH):
        parts = _sc_scatter(x, src, dst, zeros)
        x = _tc_update(parts, x, w1, w2, b2)
    return x


# speedup vs baseline: 1.9822x; 1.0811x over previous
import functools

import jax
import jax.numpy as jnp
from jax import lax
from jax.experimental import pallas as pl
from jax.experimental.pallas import tpu as pltpu
from jax.experimental.pallas import tpu_sc as plsc

N_NODES = 10000
D_FEAT = 128
N_EDGES = 320000
DEPTH = 3

NC = 2
NS = 16
NW = NC * NS

EDGES_PER_TILE = N_EDGES // NW
CHUNK = 40
CHUNKS = EDGES_PER_TILE // CHUNK
NBUF = 5
N_PAD = 10000
ROWS_PER_TILE = N_PAD // NS
ZROWS = 125

_mesh = plsc.VectorSubcoreMesh(core_axis_name="c", subcore_axis_name="s")


def _sc_body(x_hbm, src_hbm, dst_hbm, zeros_hbm, out_hbm,
             act_sh, src_v, dst_v, rows0, rows1, rows2, rows3, rows4,
             gsem0, gsem1, gsem2, gsem3, gsem4):
    c = lax.axis_index("c")
    s = lax.axis_index("s")
    wid = c * NS + s

    pltpu.sync_copy(zeros_hbm, act_sh.at[pl.ds(s * ROWS_PER_TILE, ROWS_PER_TILE)])
    pltpu.sync_copy(src_hbm.at[wid], src_v)
    pltpu.sync_copy(dst_hbm.at[wid], dst_v)
    plsc.subcore_barrier()

    rows = [rows0, rows1, rows2, rows3, rows4]
    gsems = [gsem0, gsem1, gsem2, gsem3, gsem4]
    for k in range(NBUF):
        pltpu.async_copy(x_hbm.at[src_v.at[k]], rows[k], gsems[k])

    def body(i, carry):
        j = NBUF * i
        for k in range(NBUF):
            pltpu.make_async_copy(x_hbm.at[src_v.at[j + k]], rows[k], gsems[k]).wait()
            pltpu.sync_copy(rows[k], act_sh.at[dst_v.at[j + k]], add=True)
            pltpu.async_copy(x_hbm.at[src_v.at[j + k + NBUF]], rows[k], gsems[k])
        return carry

    lax.fori_loop(0, CHUNKS // NBUF - 1, body, 0)
    for k in range(NBUF):
        j = CHUNKS - NBUF + k
        pltpu.make_async_copy(x_hbm.at[src_v.at[j]], rows[k], gsems[k]).wait()
        pltpu.sync_copy(rows[k], act_sh.at[dst_v.at[j]], add=True)
    plsc.subcore_barrier()
    sl = pl.ds(s * ROWS_PER_TILE, ROWS_PER_TILE)
    pltpu.sync_copy(act_sh.at[sl], out_hbm.at[c, sl])


_sc_scatter = pl.kernel(
    _sc_body,
    out_type=jax.ShapeDtypeStruct((NC, N_PAD, D_FEAT), jnp.float32),
    mesh=_mesh,
    compiler_params=pltpu.CompilerParams(use_tc_tiling_on_sc=False),
    scratch_types=[
        pltpu.VMEM_SHARED((N_PAD, D_FEAT), jnp.float32),
        pltpu.VMEM((CHUNKS, CHUNK), jnp.int32),
        pltpu.VMEM((CHUNKS, CHUNK), jnp.int32),
        pltpu.VMEM((CHUNK, D_FEAT), jnp.float32),
        pltpu.VMEM((CHUNK, D_FEAT), jnp.float32),
        pltpu.VMEM((CHUNK, D_FEAT), jnp.float32),
        pltpu.VMEM((CHUNK, D_FEAT), jnp.float32),
        pltpu.VMEM((CHUNK, D_FEAT), jnp.float32),
        pltpu.SemaphoreType.DMA,
        pltpu.SemaphoreType.DMA,
        pltpu.SemaphoreType.DMA,
        pltpu.SemaphoreType.DMA,
        pltpu.SemaphoreType.DMA,
    ],
)


def _tc_body(p_ref, x_ref, w1_ref, w2_ref, b_ref, o_ref):
    act = p_ref[0] + p_ref[1]
    o_ref[...] = jnp.maximum(
        jnp.dot(act, w1_ref[...], preferred_element_type=jnp.float32)
        + jnp.dot(x_ref[...], w2_ref[...], preferred_element_type=jnp.float32)
        + b_ref[...],
        0.0,
    )


_BR = 2000


@jax.jit
def _tc_update(parts, x, w1, w2, b2):
    return pl.pallas_call(
        _tc_body,
        grid=(N_NODES // _BR,),
        in_specs=[
            pl.BlockSpec((NC, _BR, D_FEAT), lambda i: (0, i, 0)),
            pl.BlockSpec((_BR, D_FEAT), lambda i: (i, 0)),
            pl.BlockSpec((D_FEAT, D_FEAT), lambda i: (0, 0)),
            pl.BlockSpec((D_FEAT, D_FEAT), lambda i: (0, 0)),
            pl.BlockSpec((1, D_FEAT), lambda i: (0, 0)),
        ],
        out_specs=pl.BlockSpec((_BR, D_FEAT), lambda i: (i, 0)),
        out_shape=jax.ShapeDtypeStruct((N_NODES, D_FEAT), jnp.float32),
    )(parts, x, w1, w2, b2)


def kernel(node_embedding, incidence, W, b):
    src = incidence[1].reshape(NW, CHUNKS, CHUNK)
    dst = incidence[0].reshape(NW, CHUNKS, CHUNK)
    w1 = W[:D_FEAT]
    w2 = W[D_FEAT:]
    b2 = b.reshape(1, D_FEAT)
    zeros = jnp.zeros((ROWS_PER_TILE, D_FEAT), jnp.float32)
    x = node_embedding
    for _ in range(DEPTH):
        parts = _sc_scatter(x, src, dst, zeros)
        x = _tc_update(parts, x, w1, w2, b2)
    return x
